# baseline (speedup 1.0000x reference)
import functools

import jax
import jax.numpy as jnp
from jax import lax
from jax.experimental import pallas as pl
from jax.experimental.pallas import tpu as pltpu
from jax.experimental.pallas import tpu_sc as plsc

N = 10000
E = 320000
DIN = 128
DH = 32
DOUT = 8
NG = 32

NC = 2
NS = 16
NW = NC * NS
CH = 128
EP = 327680
EPT = EP // NW
ITERS = EPT // CH
NP = 10240
RPT = NP // NS

BLK = 1000
GRID = N // BLK


@functools.lru_cache(maxsize=None)
def _make_sc_aggregate(W):
  mesh = plsc.VectorSubcoreMesh(core_axis_name="c", subcore_axis_name="s",
                                num_cores=NC, num_subcores=NS)

  @functools.partial(
      pl.kernel,
      out_type=jax.ShapeDtypeStruct((NC, N, W), jnp.float32),
      mesh=mesh,
      scratch_types=[
          pltpu.VMEM((ITERS, CH), jnp.int32),
          pltpu.VMEM((ITERS, CH), jnp.int32),
          pltpu.VMEM((CH, W), jnp.float32),
          pltpu.VMEM((CH, W), jnp.float32),
          pltpu.VMEM((CH, W), jnp.float32),
          pltpu.VMEM((CH, W), jnp.float32),
          pltpu.VMEM_SHARED((N, W), jnp.float32),
          pltpu.VMEM_SHARED((NP, W), jnp.float32),
          pltpu.SemaphoreType.DMA,
          pltpu.SemaphoreType.DMA,
      ],
      compiler_params=pltpu.CompilerParams(use_tc_tiling_on_sc=False),
  )
  def agg(p_hbm, src_hbm, dst_hbm, zeros_hbm, out_hbm,
          sidx, didx, rows0, rows1, rows2, rows3, p_sh, acc, gsem, ssem):
    c = lax.axis_index("c")
    s = lax.axis_index("s")
    wid = c * NS + s
    rps = N // NS
    pltpu.sync_copy(p_hbm.at[pl.ds(s * rps, rps)],
                    p_sh.at[pl.ds(s * rps, rps)])
    pltpu.sync_copy(zeros_hbm.at[pl.ds(s * rps, rps)],
                    acc.at[pl.ds(s * rps, rps)])
    row0 = wid * ITERS
    pltpu.sync_copy(src_hbm.at[pl.ds(row0, ITERS)], sidx)
    pltpu.sync_copy(dst_hbm.at[pl.ds(row0, ITERS)], didx)
    plsc.subcore_barrier()

    def start_gather(j, buf):
      pltpu.async_copy(p_sh.at[sidx.at[j]], buf, gsem)

    def start_scatter(j, buf):
      pltpu.async_copy(buf, acc.at[didx.at[j]], ssem, add=True)

    def wait_gather(buf):
      pltpu.make_async_copy(p_sh.at[sidx.at[0]], buf, gsem).wait()

    def wait_scatter(buf):
      pltpu.make_async_copy(buf, acc.at[didx.at[0]], ssem).wait()

    bufs = (rows0, rows1, rows2, rows3)
    start_gather(0, rows0)
    start_gather(1, rows1)

    def body(j4, carry):
      j = 4 * j4
      for t in range(4):
        bt = bufs[t]
        bn = bufs[(t + 2) % 4]
        wait_gather(bt)
        if t < 2:
          @pl.when(j4 > 0)
          def _(bn=bn):
            wait_scatter(bn)
        else:
          wait_scatter(bn)
        if t < 2:
          start_gather(j + t + 2, bn)
        else:
          @pl.when(j4 < ITERS // 4 - 1)
          def _(bn=bn, jg=j + t + 2):
            start_gather(jg, bn)
        start_scatter(j + t, bt)
      return carry

    lax.fori_loop(0, ITERS // 4, body, 0)
    wait_scatter(rows2)
    wait_scatter(rows3)
    plsc.subcore_barrier()
    pltpu.sync_copy(acc.at[pl.ds(s * rps, rps)],
                    out_hbm.at[c, pl.ds(s * rps, rps)])

  return agg


def _k0_body(x_ref, wl_ref, wr_ref, b_ref, paug_ref, q_ref):
  x = x_ref[...]
  p = jnp.dot(x, wl_ref[...], preferred_element_type=jnp.float32)
  one = jnp.ones((BLK, 1), jnp.float32)
  pad = jnp.zeros((BLK, 15), jnp.float32)
  paug_ref[...] = jnp.concatenate([p, one, pad], axis=1)
  q_ref[...] = jnp.dot(x, wr_ref[...], preferred_element_type=jnp.float32) + b_ref[...]


def _tc_project_in(x, Wl, Wr, b):
  return pl.pallas_call(
      _k0_body,
      grid=(GRID,),
      in_specs=[
          pl.BlockSpec((BLK, DIN), lambda i: (i, 0)),
          pl.BlockSpec((DIN, DH), lambda i: (0, 0)),
          pl.BlockSpec((DIN, DH), lambda i: (0, 0)),
          pl.BlockSpec((1, DH), lambda i: (0, 0)),
      ],
      out_specs=[
          pl.BlockSpec((BLK, DH + 16), lambda i: (i, 0)),
          pl.BlockSpec((BLK, DH), lambda i: (i, 0)),
      ],
      out_shape=[
          jax.ShapeDtypeStruct((N, DH + 16), jnp.float32),
          jax.ShapeDtypeStruct((N, DH), jnp.float32),
      ],
  )(x, Wl, Wr, b.reshape(1, DH))


def _k1_body(acc_ref, q_ref, wl_ref, wr_ref, b_ref,
             p_ref, qn_ref, cnt_ref):
  a0 = acc_ref[0]
  a1 = acc_ref[1]
  cnt = a0[:, DH:DH + 1] + a1[:, DH:DH + 1]
  agg = a0[:, :DH] + a1[:, :DH]
  h = jnp.maximum(agg / jnp.maximum(cnt, 1.0) + q_ref[...], 0.0)
  p_ref[...] = jnp.dot(h, wl_ref[...], preferred_element_type=jnp.float32)
  qn_ref[...] = jnp.dot(h, wr_ref[...], preferred_element_type=jnp.float32) + b_ref[...]
  cnt_ref[...] = cnt


def _tc_combine1(acc, q, Wl, Wr, b):
  return pl.pallas_call(
      _k1_body,
      grid=(GRID,),
      in_specs=[
          pl.BlockSpec((NC, BLK, DH + 16), lambda i: (0, i, 0)),
          pl.BlockSpec((BLK, DH), lambda i: (i, 0)),
          pl.BlockSpec((DH, DH), lambda i: (0, 0)),
          pl.BlockSpec((DH, DH), lambda i: (0, 0)),
          pl.BlockSpec((1, DH), lambda i: (0, 0)),
      ],
      out_specs=[
          pl.BlockSpec((BLK, DH), lambda i: (i, 0)),
          pl.BlockSpec((BLK, DH), lambda i: (i, 0)),
          pl.BlockSpec((BLK, 1), lambda i: (i, 0)),
      ],
      out_shape=[
          jax.ShapeDtypeStruct((N, DH), jnp.float32),
          jax.ShapeDtypeStruct((N, DH), jnp.float32),
          jax.ShapeDtypeStruct((N, 1), jnp.float32),
      ],
  )(acc, q, Wl, Wr, b.reshape(1, DH))


def _k2_body(acc_ref, q_ref, cnt_ref, wl_ref, wr_ref, b_ref,
             p_ref, qn_ref):
  agg = acc_ref[0] + acc_ref[1]
  h = jnp.maximum(agg / jnp.maximum(cnt_ref[...], 1.0) + q_ref[...], 0.0)
  p_ref[...] = jnp.dot(h, wl_ref[...], preferred_element_type=jnp.float32)
  qn_ref[...] = jnp.dot(h, wr_ref[...], preferred_element_type=jnp.float32) + b_ref[...]


def _tc_combine2(acc, q, cnt, Wl, Wr, b):
  return pl.pallas_call(
      _k2_body,
      grid=(GRID,),
      in_specs=[
          pl.BlockSpec((NC, BLK, DH), lambda i: (0, i, 0)),
          pl.BlockSpec((BLK, DH), lambda i: (i, 0)),
          pl.BlockSpec((BLK, 1), lambda i: (i, 0)),
          pl.BlockSpec((DH, DH), lambda i: (0, 0)),
          pl.BlockSpec((DH, DH), lambda i: (0, 0)),
          pl.BlockSpec((1, DH), lambda i: (0, 0)),
      ],
      out_specs=[
          pl.BlockSpec((BLK, DH), lambda i: (i, 0)),
          pl.BlockSpec((BLK, DH), lambda i: (i, 0)),
      ],
      out_shape=[
          jax.ShapeDtypeStruct((N, DH), jnp.float32),
          jax.ShapeDtypeStruct((N, DH), jnp.float32),
      ],
  )(acc, q, cnt, Wl, Wr, b.reshape(1, DH))


def _k3_body(acc_ref, q_ref, cnt_ref, batch_ref,
             l1w_ref, l1b_ref, l2w_ref, l2b_ref, out_ref,
             gs_ref, gc_ref):
  i = pl.program_id(0)

  @pl.when(i == 0)
  def _():
    gs_ref[...] = jnp.zeros_like(gs_ref)
    gc_ref[...] = jnp.zeros_like(gc_ref)

  agg = acc_ref[0] + acc_ref[1]
  h = jnp.maximum(agg / jnp.maximum(cnt_ref[...], 1.0) + q_ref[...], 0.0)
  groups = lax.broadcasted_iota(jnp.int32, (BLK, NG), 1)
  onehot = (batch_ref[...] == groups).astype(jnp.float32)
  gs_ref[...] += lax.dot_general(
      onehot, h, (((0,), (0,)), ((), ())),
      preferred_element_type=jnp.float32)
  gc_ref[...] += lax.dot_general(
      onehot, jnp.ones((BLK, DH), jnp.float32), (((0,), (0,)), ((), ())),
      preferred_element_type=jnp.float32)

  @pl.when(i == GRID - 1)
  def _():
    g = gs_ref[...] / jnp.maximum(gc_ref[...], 1.0)
    z = jnp.maximum(
        jnp.dot(g, l1w_ref[...], preferred_element_type=jnp.float32)
        + l1b_ref[...], 0.0)
    out_ref[...] = jnp.dot(
        z, l2w_ref[...], preferred_element_type=jnp.float32) + l2b_ref[...]


def _tc_head(acc, q, cnt, batch2d, L1W, L1b, L2W, L2b):
  return pl.pallas_call(
      _k3_body,
      grid=(GRID,),
      in_specs=[
          pl.BlockSpec((NC, BLK, DH), lambda i: (0, i, 0)),
          pl.BlockSpec((BLK, DH), lambda i: (i, 0)),
          pl.BlockSpec((BLK, 1), lambda i: (i, 0)),
          pl.BlockSpec((BLK, 1), lambda i: (i, 0)),
          pl.BlockSpec((DH, DH), lambda i: (0, 0)),
          pl.BlockSpec((1, DH), lambda i: (0, 0)),
          pl.BlockSpec((DH, DOUT), lambda i: (0, 0)),
          pl.BlockSpec((1, DOUT), lambda i: (0, 0)),
      ],
      out_specs=pl.BlockSpec((NG, DOUT), lambda i: (0, 0)),
      out_shape=jax.ShapeDtypeStruct((NG, DOUT), jnp.float32),
      scratch_shapes=[
          pltpu.VMEM((NG, DH), jnp.float32),
          pltpu.VMEM((NG, DH), jnp.float32),
      ],
  )(acc, q, cnt, batch2d, L1W, L1b.reshape(1, DH), L2W, L2b.reshape(1, DOUT))


def kernel(x, edge_index, edge_weight, batch,
           W1l, W1r, b1, W2l, W2r, b2, W3l, W3r, b3,
           L1W, L1b, L2W, L2b):
  del edge_weight
  pad = EP - E
  src = jnp.concatenate(
      [edge_index[0], jnp.zeros((pad,), jnp.int32)]).reshape(EP // CH, CH)
  dst = jnp.concatenate(
      [edge_index[1], jnp.full((pad,), NP - 1, jnp.int32)]).reshape(EP // CH, CH)
  zeros48 = jnp.zeros((N, DH + 16), jnp.float32)
  zeros32 = jnp.zeros((N, DH), jnp.float32)

  p1, q1 = _tc_project_in(x, W1l, W1r, b1)
  acc1 = _make_sc_aggregate(DH + 16)(p1, src, dst, zeros48)
  p2, q2, cnt = _tc_combine1(acc1, q1, W2l, W2r, b2)
  acc2 = _make_sc_aggregate(DH)(p2, src, dst, zeros32)
  p3, q3 = _tc_combine2(acc2, q2, cnt, W3l, W3r, b3)
  acc3 = _make_sc_aggregate(DH)(p3, src, dst, zeros32)
  out = _tc_head(acc3, q3, cnt, batch.reshape(N, 1),
                 L1W, L1b, L2W, L2b)
  return out


# speedup vs baseline: 1.0114x; 1.0114x over previous
import functools

import jax
import jax.numpy as jnp
from jax import lax
from jax.experimental import pallas as pl
from jax.experimental.pallas import tpu as pltpu
from jax.experimental.pallas import tpu_sc as plsc

N = 10000
E = 320000
DIN = 128
DH = 32
DOUT = 8
NG = 32

NC = 2
NS = 16
NW = NC * NS
CH = 128
EP = 327680
EPT = EP // NW
ITERS = EPT // CH
NP = 10240
RPT = NP // NS

BLK = 2000
GRID = N // BLK


@functools.lru_cache(maxsize=None)
def _make_sc_aggregate(W, with_cnt=False):
  mesh = plsc.VectorSubcoreMesh(core_axis_name="c", subcore_axis_name="s",
                                num_cores=NC, num_subcores=NS)
  out_type = [jax.ShapeDtypeStruct((NC, N, W), jnp.float32)]
  scratch = [
      pltpu.VMEM((ITERS, CH), jnp.int32),
      pltpu.VMEM((ITERS, CH), jnp.int32),
      pltpu.VMEM((CH, W), jnp.float32),
      pltpu.VMEM((CH, W), jnp.float32),
      pltpu.VMEM((CH, W), jnp.float32),
      pltpu.VMEM((CH, W), jnp.float32),
      pltpu.VMEM_SHARED((N, W), jnp.float32),
      pltpu.VMEM_SHARED((NP, W), jnp.float32),
      pltpu.SemaphoreType.DMA,
      pltpu.SemaphoreType.DMA,
  ]
  if with_cnt:
    out_type.append(jax.ShapeDtypeStruct((NC, N, 16), jnp.float32))
    scratch += [
        pltpu.VMEM((CH, 16), jnp.float32),
        pltpu.VMEM_SHARED((NP, 16), jnp.float32),
        pltpu.SemaphoreType.DMA,
    ]

  def agg(p_hbm, src_hbm, dst_hbm, zeros_hbm, *rest):
    if with_cnt:
      (ones_hbm, zeros16_hbm, out_hbm, cnt_hbm,
       sidx, didx, rows0, rows1, rows2, rows3, p_sh, acc,
       gsem, ssem, ones_vm, cnt_sh, csem) = rest
    else:
      (out_hbm,
       sidx, didx, rows0, rows1, rows2, rows3, p_sh, acc,
       gsem, ssem) = rest
    c = lax.axis_index("c")
    s = lax.axis_index("s")
    wid = c * NS + s
    rps = N // NS
    pltpu.sync_copy(p_hbm.at[pl.ds(s * rps, rps)],
                    p_sh.at[pl.ds(s * rps, rps)])
    pltpu.sync_copy(zeros_hbm.at[pl.ds(s * rps, rps)],
                    acc.at[pl.ds(s * rps, rps)])
    if with_cnt:
      pltpu.sync_copy(ones_hbm, ones_vm)
      pltpu.sync_copy(zeros16_hbm.at[pl.ds(s * rps, rps)],
                      cnt_sh.at[pl.ds(s * rps, rps)])
    row0 = wid * ITERS
    pltpu.sync_copy(src_hbm.at[pl.ds(row0, ITERS)], sidx)
    pltpu.sync_copy(dst_hbm.at[pl.ds(row0, ITERS)], didx)
    plsc.subcore_barrier()

    def start_gather(j, buf):
      pltpu.async_copy(p_sh.at[sidx.at[j]], buf, gsem)

    def start_scatter(j, buf):
      pltpu.async_copy(buf, acc.at[didx.at[j]], ssem, add=True)
      if with_cnt:
        pltpu.async_copy(ones_vm, cnt_sh.at[didx.at[j]], csem, add=True)

    def wait_gather(buf):
      pltpu.make_async_copy(p_sh.at[sidx.at[0]], buf, gsem).wait()

    def wait_scatter(buf):
      pltpu.make_async_copy(buf, acc.at[didx.at[0]], ssem).wait()

    bufs = (rows0, rows1, rows2, rows3)
    start_gather(0, rows0)
    start_gather(1, rows1)

    def body(j4, carry):
      j = 4 * j4
      for t in range(4):
        bt = bufs[t]
        bn = bufs[(t + 2) % 4]
        wait_gather(bt)
        if t < 2:
          @pl.when(j4 > 0)
          def _(bn=bn):
            wait_scatter(bn)
        else:
          wait_scatter(bn)
        if t < 2:
          start_gather(j + t + 2, bn)
        else:
          @pl.when(j4 < ITERS // 4 - 1)
          def _(bn=bn, jg=j + t + 2):
            start_gather(jg, bn)
        start_scatter(j + t, bt)
      return carry

    lax.fori_loop(0, ITERS // 4, body, 0)
    wait_scatter(rows2)
    wait_scatter(rows3)
    if with_cnt:
      def drain(j, carry):
        pltpu.make_async_copy(ones_vm, cnt_sh.at[didx.at[0]], csem).wait()
        return carry
      lax.fori_loop(0, ITERS, drain, 0)
    plsc.subcore_barrier()
    pltpu.sync_copy(acc.at[pl.ds(s * rps, rps)],
                    out_hbm.at[c, pl.ds(s * rps, rps)])
    if with_cnt:
      pltpu.sync_copy(cnt_sh.at[pl.ds(s * rps, rps)],
                      cnt_hbm.at[c, pl.ds(s * rps, rps)])

  return pl.kernel(
      agg,
      out_type=out_type if with_cnt else out_type[0],
      mesh=mesh,
      scratch_types=scratch,
      compiler_params=pltpu.CompilerParams(use_tc_tiling_on_sc=False),
  )


def _k0_body(x_ref, wl_ref, wr_ref, b_ref, p_ref, q_ref):
  x = x_ref[...]
  p_ref[...] = jnp.dot(x, wl_ref[...], preferred_element_type=jnp.float32)
  q_ref[...] = jnp.dot(x, wr_ref[...], preferred_element_type=jnp.float32) + b_ref[...]


def _tc_project_in(x, Wl, Wr, b):
  return pl.pallas_call(
      _k0_body,
      grid=(GRID,),
      in_specs=[
          pl.BlockSpec((BLK, DIN), lambda i: (i, 0)),
          pl.BlockSpec((DIN, DH), lambda i: (0, 0)),
          pl.BlockSpec((DIN, DH), lambda i: (0, 0)),
          pl.BlockSpec((1, DH), lambda i: (0, 0)),
      ],
      out_specs=[
          pl.BlockSpec((BLK, DH), lambda i: (i, 0)),
          pl.BlockSpec((BLK, DH), lambda i: (i, 0)),
      ],
      out_shape=[
          jax.ShapeDtypeStruct((N, DH), jnp.float32),
          jax.ShapeDtypeStruct((N, DH), jnp.float32),
      ],
  )(x, Wl, Wr, b.reshape(1, DH))


def _k1_body(acc_ref, cp_ref, q_ref, wl_ref, wr_ref, b_ref,
             p_ref, qn_ref, cnt_ref):
  cnt = cp_ref[0, :, 0:1] + cp_ref[1, :, 0:1]
  agg = acc_ref[0] + acc_ref[1]
  h = jnp.maximum(agg / jnp.maximum(cnt, 1.0) + q_ref[...], 0.0)
  p_ref[...] = jnp.dot(h, wl_ref[...], preferred_element_type=jnp.float32)
  qn_ref[...] = jnp.dot(h, wr_ref[...], preferred_element_type=jnp.float32) + b_ref[...]
  cnt_ref[...] = cnt


def _tc_combine1(acc, cntp, q, Wl, Wr, b):
  return pl.pallas_call(
      _k1_body,
      grid=(GRID,),
      in_specs=[
          pl.BlockSpec((NC, BLK, DH), lambda i: (0, i, 0)),
          pl.BlockSpec((NC, BLK, 16), lambda i: (0, i, 0)),
          pl.BlockSpec((BLK, DH), lambda i: (i, 0)),
          pl.BlockSpec((DH, DH), lambda i: (0, 0)),
          pl.BlockSpec((DH, DH), lambda i: (0, 0)),
          pl.BlockSpec((1, DH), lambda i: (0, 0)),
      ],
      out_specs=[
          pl.BlockSpec((BLK, DH), lambda i: (i, 0)),
          pl.BlockSpec((BLK, DH), lambda i: (i, 0)),
          pl.BlockSpec((BLK, 1), lambda i: (i, 0)),
      ],
      out_shape=[
          jax.ShapeDtypeStruct((N, DH), jnp.float32),
          jax.ShapeDtypeStruct((N, DH), jnp.float32),
          jax.ShapeDtypeStruct((N, 1), jnp.float32),
      ],
  )(acc, cntp, q, Wl, Wr, b.reshape(1, DH))


def _k2_body(acc_ref, q_ref, cnt_ref, wl_ref, wr_ref, b_ref,
             p_ref, qn_ref):
  agg = acc_ref[0] + acc_ref[1]
  h = jnp.maximum(agg / jnp.maximum(cnt_ref[...], 1.0) + q_ref[...], 0.0)
  p_ref[...] = jnp.dot(h, wl_ref[...], preferred_element_type=jnp.float32)
  qn_ref[...] = jnp.dot(h, wr_ref[...], preferred_element_type=jnp.float32) + b_ref[...]


def _tc_combine2(acc, q, cnt, Wl, Wr, b):
  return pl.pallas_call(
      _k2_body,
      grid=(GRID,),
      in_specs=[
          pl.BlockSpec((NC, BLK, DH), lambda i: (0, i, 0)),
          pl.BlockSpec((BLK, DH), lambda i: (i, 0)),
          pl.BlockSpec((BLK, 1), lambda i: (i, 0)),
          pl.BlockSpec((DH, DH), lambda i: (0, 0)),
          pl.BlockSpec((DH, DH), lambda i: (0, 0)),
          pl.BlockSpec((1, DH), lambda i: (0, 0)),
      ],
      out_specs=[
          pl.BlockSpec((BLK, DH), lambda i: (i, 0)),
          pl.BlockSpec((BLK, DH), lambda i: (i, 0)),
      ],
      out_shape=[
          jax.ShapeDtypeStruct((N, DH), jnp.float32),
          jax.ShapeDtypeStruct((N, DH), jnp.float32),
      ],
  )(acc, q, cnt, Wl, Wr, b.reshape(1, DH))


def _k3_body(acc_ref, q_ref, cnt_ref, batch_ref,
             l1w_ref, l1b_ref, l2w_ref, l2b_ref, out_ref,
             gs_ref, gc_ref):
  i = pl.program_id(0)

  @pl.when(i == 0)
  def _():
    gs_ref[...] = jnp.zeros_like(gs_ref)
    gc_ref[...] = jnp.zeros_like(gc_ref)

  agg = acc_ref[0] + acc_ref[1]
  h = jnp.maximum(agg / jnp.maximum(cnt_ref[...], 1.0) + q_ref[...], 0.0)
  groups = lax.broadcasted_iota(jnp.int32, (BLK, NG), 1)
  onehot = (batch_ref[...] == groups).astype(jnp.float32)
  gs_ref[...] += lax.dot_general(
      onehot, h, (((0,), (0,)), ((), ())),
      preferred_element_type=jnp.float32)
  gc_ref[...] += lax.dot_general(
      onehot, jnp.ones((BLK, DH), jnp.float32), (((0,), (0,)), ((), ())),
      preferred_element_type=jnp.float32)

  @pl.when(i == GRID - 1)
  def _():
    g = gs_ref[...] / jnp.maximum(gc_ref[...], 1.0)
    z = jnp.maximum(
        jnp.dot(g, l1w_ref[...], preferred_element_type=jnp.float32)
        + l1b_ref[...], 0.0)
    out_ref[...] = jnp.dot(
        z, l2w_ref[...], preferred_element_type=jnp.float32) + l2b_ref[...]


def _tc_head(acc, q, cnt, batch2d, L1W, L1b, L2W, L2b):
  return pl.pallas_call(
      _k3_body,
      grid=(GRID,),
      in_specs=[
          pl.BlockSpec((NC, BLK, DH), lambda i: (0, i, 0)),
          pl.BlockSpec((BLK, DH), lambda i: (i, 0)),
          pl.BlockSpec((BLK, 1), lambda i: (i, 0)),
          pl.BlockSpec((BLK, 1), lambda i: (i, 0)),
          pl.BlockSpec((DH, DH), lambda i: (0, 0)),
          pl.BlockSpec((1, DH), lambda i: (0, 0)),
          pl.BlockSpec((DH, DOUT), lambda i: (0, 0)),
          pl.BlockSpec((1, DOUT), lambda i: (0, 0)),
      ],
      out_specs=pl.BlockSpec((NG, DOUT), lambda i: (0, 0)),
      out_shape=jax.ShapeDtypeStruct((NG, DOUT), jnp.float32),
      scratch_shapes=[
          pltpu.VMEM((NG, DH), jnp.float32),
          pltpu.VMEM((NG, DH), jnp.float32),
      ],
  )(acc, q, cnt, batch2d, L1W, L1b.reshape(1, DH), L2W, L2b.reshape(1, DOUT))


def kernel(x, edge_index, edge_weight, batch,
           W1l, W1r, b1, W2l, W2r, b2, W3l, W3r, b3,
           L1W, L1b, L2W, L2b):
  del edge_weight
  pad = EP - E
  src = jnp.concatenate(
      [edge_index[0], jnp.zeros((pad,), jnp.int32)]).reshape(EP // CH, CH)
  dst = jnp.concatenate(
      [edge_index[1], jnp.full((pad,), NP - 1, jnp.int32)]).reshape(EP // CH, CH)
  zeros32 = jnp.zeros((N, DH), jnp.float32)
  zeros16 = jnp.zeros((N, 16), jnp.float32)
  ones16 = jnp.ones((CH, 16), jnp.float32)

  p1, q1 = _tc_project_in(x, W1l, W1r, b1)
  acc1, cntp = _make_sc_aggregate(DH, True)(p1, src, dst, zeros32,
                                            ones16, zeros16)
  p2, q2, cnt = _tc_combine1(acc1, cntp, q1, W2l, W2r, b2)
  acc2 = _make_sc_aggregate(DH)(p2, src, dst, zeros32)
  p3, q3 = _tc_combine2(acc2, q2, cnt, W3l, W3r, b3)
  acc3 = _make_sc_aggregate(DH)(p3, src, dst, zeros32)
  out = _tc_head(acc3, q3, cnt, batch.reshape(N, 1),
                 L1W, L1b, L2W, L2b)
  return out


# speedup vs baseline: 1.0380x; 1.0263x over previous
import functools

import jax
import jax.numpy as jnp
from jax import lax
from jax.experimental import pallas as pl
from jax.experimental.pallas import tpu as pltpu
from jax.experimental.pallas import tpu_sc as plsc

N = 10000
E = 320000
DIN = 128
DH = 32
DOUT = 8
NG = 32

NC = 2
NS = 16
NW = NC * NS
CH = 128
EP = 327680
EPT = EP // NW
ITERS = EPT // CH
NP = 10240
RPT = NP // NS

BLK = 2000
GRID = N // BLK


@functools.lru_cache(maxsize=None)
def _make_sc_aggregate(W):
  mesh = plsc.VectorSubcoreMesh(core_axis_name="c", subcore_axis_name="s",
                                num_cores=NC, num_subcores=NS)

  @functools.partial(
      pl.kernel,
      out_type=jax.ShapeDtypeStruct((NC, N, W), jnp.float32),
      mesh=mesh,
      scratch_types=[
          pltpu.VMEM((ITERS, CH), jnp.int32),
          pltpu.VMEM((ITERS, CH), jnp.int32),
          pltpu.VMEM((CH, W), jnp.float32),
          pltpu.VMEM((CH, W), jnp.float32),
          pltpu.VMEM((CH, W), jnp.float32),
          pltpu.VMEM((CH, W), jnp.float32),
          pltpu.VMEM_SHARED((N, W), jnp.float32),
          pltpu.VMEM_SHARED((NP, W), jnp.float32),
          pltpu.SemaphoreType.DMA,
          pltpu.SemaphoreType.DMA,
      ],
      compiler_params=pltpu.CompilerParams(use_tc_tiling_on_sc=False),
  )
  def agg(p_hbm, src_hbm, dst_hbm, zeros_hbm, out_hbm,
          sidx, didx, rows0, rows1, rows2, rows3, p_sh, acc, gsem, ssem):
    c = lax.axis_index("c")
    s = lax.axis_index("s")
    wid = c * NS + s
    rps = N // NS
    pltpu.sync_copy(p_hbm.at[pl.ds(s * rps, rps)],
                    p_sh.at[pl.ds(s * rps, rps)])
    pltpu.sync_copy(zeros_hbm.at[pl.ds(s * rps, rps)],
                    acc.at[pl.ds(s * rps, rps)])
    row0 = wid * ITERS
    pltpu.sync_copy(src_hbm.at[pl.ds(row0, ITERS)], sidx)
    pltpu.sync_copy(dst_hbm.at[pl.ds(row0, ITERS)], didx)
    plsc.subcore_barrier()

    def start_gather(j, buf):
      pltpu.async_copy(p_sh.at[sidx.at[j]], buf, gsem)

    def start_scatter(j, buf):
      pltpu.async_copy(buf, acc.at[didx.at[j]], ssem, add=True)

    def wait_gather(buf):
      pltpu.make_async_copy(p_sh.at[sidx.at[0]], buf, gsem).wait()

    def wait_scatter(buf):
      pltpu.make_async_copy(buf, acc.at[didx.at[0]], ssem).wait()

    bufs = (rows0, rows1, rows2, rows3)
    start_gather(0, rows0)
    start_gather(1, rows1)

    def body(j4, carry):
      j = 4 * j4
      for t in range(4):
        bt = bufs[t]
        bn = bufs[(t + 2) % 4]
        wait_gather(bt)
        if t < 2:
          @pl.when(j4 > 0)
          def _(bn=bn):
            wait_scatter(bn)
        else:
          wait_scatter(bn)
        if t < 2:
          start_gather(j + t + 2, bn)
        else:
          @pl.when(j4 < ITERS // 4 - 1)
          def _(bn=bn, jg=j + t + 2):
            start_gather(jg, bn)
        start_scatter(j + t, bt)
      return carry

    lax.fori_loop(0, ITERS // 4, body, 0)
    wait_scatter(rows2)
    wait_scatter(rows3)
    plsc.subcore_barrier()
    pltpu.sync_copy(acc.at[pl.ds(s * rps, rps)],
                    out_hbm.at[c, pl.ds(s * rps, rps)])

  return agg


def _k0_body(x_ref, wl_ref, wr_ref, b_ref, paug_ref, q_ref):
  x = x_ref[...]
  p = jnp.dot(x, wl_ref[...], preferred_element_type=jnp.float32)
  one = jnp.ones((BLK, 1), jnp.float32)
  pad = jnp.zeros((BLK, 15), jnp.float32)
  paug_ref[...] = jnp.concatenate([p, one, pad], axis=1)
  q_ref[...] = jnp.dot(x, wr_ref[...], preferred_element_type=jnp.float32) + b_ref[...]


def _tc_project_in(x, Wl, Wr, b):
  return pl.pallas_call(
      _k0_body,
      grid=(GRID,),
      in_specs=[
          pl.BlockSpec((BLK, DIN), lambda i: (i, 0)),
          pl.BlockSpec((DIN, DH), lambda i: (0, 0)),
          pl.BlockSpec((DIN, DH), lambda i: (0, 0)),
          pl.BlockSpec((1, DH), lambda i: (0, 0)),
      ],
      out_specs=[
          pl.BlockSpec((BLK, DH + 16), lambda i: (i, 0)),
          pl.BlockSpec((BLK, DH), lambda i: (i, 0)),
      ],
      out_shape=[
          jax.ShapeDtypeStruct((N, DH + 16), jnp.float32),
          jax.ShapeDtypeStruct((N, DH), jnp.float32),
      ],
  )(x, Wl, Wr, b.reshape(1, DH))


def _k1_body(acc_ref, q_ref, wl_ref, wr_ref, b_ref,
             p_ref, qn_ref, cnt_ref):
  a0 = acc_ref[0]
  a1 = acc_ref[1]
  cnt = a0[:, DH:DH + 1] + a1[:, DH:DH + 1]
  agg = a0[:, :DH] + a1[:, :DH]
  h = jnp.maximum(agg / jnp.maximum(cnt, 1.0) + q_ref[...], 0.0)
  p_ref[...] = jnp.dot(h, wl_ref[...], preferred_element_type=jnp.float32)
  qn_ref[...] = jnp.dot(h, wr_ref[...], preferred_element_type=jnp.float32) + b_ref[...]
  cnt_ref[...] = cnt


def _tc_combine1(acc, q, Wl, Wr, b):
  return pl.pallas_call(
      _k1_body,
      grid=(GRID,),
      in_specs=[
          pl.BlockSpec((NC, BLK, DH + 16), lambda i: (0, i, 0)),
          pl.BlockSpec((BLK, DH), lambda i: (i, 0)),
          pl.BlockSpec((DH, DH), lambda i: (0, 0)),
          pl.BlockSpec((DH, DH), lambda i: (0, 0)),
          pl.BlockSpec((1, DH), lambda i: (0, 0)),
      ],
      out_specs=[
          pl.BlockSpec((BLK, DH), lambda i: (i, 0)),
          pl.BlockSpec((BLK, DH), lambda i: (i, 0)),
          pl.BlockSpec((BLK, 1), lambda i: (i, 0)),
      ],
      out_shape=[
          jax.ShapeDtypeStruct((N, DH), jnp.float32),
          jax.ShapeDtypeStruct((N, DH), jnp.float32),
          jax.ShapeDtypeStruct((N, 1), jnp.float32),
      ],
  )(acc, q, Wl, Wr, b.reshape(1, DH))


def _k2_body(acc_ref, q_ref, cnt_ref, wl_ref, wr_ref, b_ref,
             p_ref, qn_ref):
  agg = acc_ref[0] + acc_ref[1]
  h = jnp.maximum(agg / jnp.maximum(cnt_ref[...], 1.0) + q_ref[...], 0.0)
  p_ref[...] = jnp.dot(h, wl_ref[...], preferred_element_type=jnp.float32)
  qn_ref[...] = jnp.dot(h, wr_ref[...], preferred_element_type=jnp.float32) + b_ref[...]


def _tc_combine2(acc, q, cnt, Wl, Wr, b):
  return pl.pallas_call(
      _k2_body,
      grid=(GRID,),
      in_specs=[
          pl.BlockSpec((NC, BLK, DH), lambda i: (0, i, 0)),
          pl.BlockSpec((BLK, DH), lambda i: (i, 0)),
          pl.BlockSpec((BLK, 1), lambda i: (i, 0)),
          pl.BlockSpec((DH, DH), lambda i: (0, 0)),
          pl.BlockSpec((DH, DH), lambda i: (0, 0)),
          pl.BlockSpec((1, DH), lambda i: (0, 0)),
      ],
      out_specs=[
          pl.BlockSpec((BLK, DH), lambda i: (i, 0)),
          pl.BlockSpec((BLK, DH), lambda i: (i, 0)),
      ],
      out_shape=[
          jax.ShapeDtypeStruct((N, DH), jnp.float32),
          jax.ShapeDtypeStruct((N, DH), jnp.float32),
      ],
  )(acc, q, cnt, Wl, Wr, b.reshape(1, DH))


def _k3_body(acc_ref, q_ref, cnt_ref, batch_ref,
             l1w_ref, l1b_ref, l2w_ref, l2b_ref, out_ref,
             gs_ref, gc_ref):
  i = pl.program_id(0)

  @pl.when(i == 0)
  def _():
    gs_ref[...] = jnp.zeros_like(gs_ref)
    gc_ref[...] = jnp.zeros_like(gc_ref)

  agg = acc_ref[0] + acc_ref[1]
  h = jnp.maximum(agg / jnp.maximum(cnt_ref[...], 1.0) + q_ref[...], 0.0)
  groups = lax.broadcasted_iota(jnp.int32, (BLK, NG), 1)
  onehot = (batch_ref[...] == groups).astype(jnp.float32)
  gs_ref[...] += lax.dot_general(
      onehot, h, (((0,), (0,)), ((), ())),
      preferred_element_type=jnp.float32)
  gc_ref[...] += lax.dot_general(
      onehot, jnp.ones((BLK, DH), jnp.float32), (((0,), (0,)), ((), ())),
      preferred_element_type=jnp.float32)

  @pl.when(i == GRID - 1)
  def _():
    g = gs_ref[...] / jnp.maximum(gc_ref[...], 1.0)
    z = jnp.maximum(
        jnp.dot(g, l1w_ref[...], preferred_element_type=jnp.float32)
        + l1b_ref[...], 0.0)
    out_ref[...] = jnp.dot(
        z, l2w_ref[...], preferred_element_type=jnp.float32) + l2b_ref[...]


def _tc_head(acc, q, cnt, batch2d, L1W, L1b, L2W, L2b):
  return pl.pallas_call(
      _k3_body,
      grid=(GRID,),
      in_specs=[
          pl.BlockSpec((NC, BLK, DH), lambda i: (0, i, 0)),
          pl.BlockSpec((BLK, DH), lambda i: (i, 0)),
          pl.BlockSpec((BLK, 1), lambda i: (i, 0)),
          pl.BlockSpec((BLK, 1), lambda i: (i, 0)),
          pl.BlockSpec((DH, DH), lambda i: (0, 0)),
          pl.BlockSpec((1, DH), lambda i: (0, 0)),
          pl.BlockSpec((DH, DOUT), lambda i: (0, 0)),
          pl.BlockSpec((1, DOUT), lambda i: (0, 0)),
      ],
      out_specs=pl.BlockSpec((NG, DOUT), lambda i: (0, 0)),
      out_shape=jax.ShapeDtypeStruct((NG, DOUT), jnp.float32),
      scratch_shapes=[
          pltpu.VMEM((NG, DH), jnp.float32),
          pltpu.VMEM((NG, DH), jnp.float32),
      ],
  )(acc, q, cnt, batch2d, L1W, L1b.reshape(1, DH), L2W, L2b.reshape(1, DOUT))


def kernel(x, edge_index, edge_weight, batch,
           W1l, W1r, b1, W2l, W2r, b2, W3l, W3r, b3,
           L1W, L1b, L2W, L2b):
  del edge_weight
  pad = EP - E
  src = jnp.concatenate(
      [edge_index[0], jnp.zeros((pad,), jnp.int32)]).reshape(EP // CH, CH)
  dst = jnp.concatenate(
      [edge_index[1], jnp.full((pad,), NP - 1, jnp.int32)]).reshape(EP // CH, CH)
  zeros48 = jnp.zeros((N, DH + 16), jnp.float32)
  zeros32 = jnp.zeros((N, DH), jnp.float32)

  p1, q1 = _tc_project_in(x, W1l, W1r, b1)
  acc1 = _make_sc_aggregate(DH + 16)(p1, src, dst, zeros48)
  p2, q2, cnt = _tc_combine1(acc1, q1, W2l, W2r, b2)
  acc2 = _make_sc_aggregate(DH)(p2, src, dst, zeros32)
  p3, q3 = _tc_combine2(acc2, q2, cnt, W3l, W3r, b3)
  acc3 = _make_sc_aggregate(DH)(p3, src, dst, zeros32)
  out = _tc_head(acc3, q3, cnt, batch.reshape(N, 1),
                 L1W, L1b, L2W, L2b)
  return out
